# baseline (speedup 1.0000x reference)
import functools

import jax
import jax.numpy as jnp
from jax import lax
from jax.experimental import pallas as pl
from jax.experimental.pallas import tpu as pltpu
from jax.experimental.pallas import tpu_sc as plsc

_NC = 2
_NS = 16
_NW = _NC * _NS
_L = 16


def _make_sc_partials(n_neurons, n_edges, batch, chunk):
    slices = _NW // batch
    epw = n_edges // slices
    chunks_pw = epw // chunk
    assert epw * slices == n_edges and chunks_pw * chunk == epw
    assert chunk % _L == 0 and chunk % 8 == 0 and chunks_pw % 2 == 0

    mesh = plsc.VectorSubcoreMesh(
        core_axis_name="c", subcore_axis_name="s",
        num_cores=_NC, num_subcores=_NS)

    @functools.partial(
        pl.kernel,
        out_type=jax.ShapeDtypeStruct((_NW * n_neurons,), jnp.float32),
        mesh=mesh,
        scratch_types=[
            pltpu.VMEM((n_neurons,), jnp.float32),
            pltpu.VMEM((n_neurons,), jnp.float32),
            pltpu.VMEM((chunk,), jnp.int32),
            pltpu.VMEM((chunk,), jnp.int32),
            pltpu.VMEM((chunk,), jnp.float32),
            pltpu.VMEM((chunk,), jnp.float32),
            pltpu.SemaphoreType.DMA,
            pltpu.SemaphoreType.DMA,
            pltpu.SemaphoreType.DMA,
        ],
        compiler_params=pltpu.CompilerParams(needs_layout_passes=False),
    )
    def sc_partials(z_hbm, pp_hbm, w_hbm, part_hbm,
                    z_v, acc_v, pp0, pp1, w0, w1,
                    sem0, sem1, semz):
        wid = lax.axis_index("s") * _NC + lax.axis_index("c")
        b = wid % batch
        s = wid // batch
        zcopy = pltpu.async_copy(
            z_hbm.at[pl.ds(b * n_neurons, n_neurons)], z_v, semz)

        zero = jnp.zeros((_L,), jnp.float32)

        @plsc.parallel_loop(0, n_neurons // _L, unroll=8)
        def _(i):
            acc_v[pl.ds(i * _L, _L)] = zero
        zcopy.wait()

        base = s * chunks_pw
        bufs = ((pp0, w0), (pp1, w1))
        sems = (sem0, sem1)

        def start(g, slot):
            off = (base + g) * chunk
            pv, wv = bufs[slot]
            pltpu.async_copy(pp_hbm.at[pl.ds(off, chunk)], pv, sems[slot])
            pltpu.async_copy(w_hbm.at[pl.ds(off, chunk)], wv, sems[slot])

        def drain(g, slot):
            off = (base + g) * chunk
            pv, wv = bufs[slot]
            pltpu.make_async_copy(pp_hbm.at[pl.ds(off, chunk)], pv,
                                  sems[slot]).wait()
            pltpu.make_async_copy(w_hbm.at[pl.ds(off, chunk)], wv,
                                  sems[slot]).wait()

        start(0, 0)
        start(1, 1)

        def pair_body(gp, carry):
            for slot in range(2):
                g = gp * 2 + slot
                drain(g, slot)
                pv, wv = bufs[slot]

                @plsc.parallel_loop(0, chunk // _L, unroll=10)
                def _(j):
                    sl = pl.ds(j * _L, _L)
                    pp = pv[sl]
                    w = wv[sl]
                    pre = lax.shift_right_logical(pp, 16)
                    post = lax.bitwise_and(pp, 0xFFFF)
                    zg = plsc.load_gather(z_v, [pre])
                    plsc.addupdate_scatter(acc_v, [post], zg * w)

                @pl.when(g + 2 < chunks_pw)
                def _():
                    start(g + 2, slot)
            return carry
        lax.fori_loop(0, chunks_pw // 2, pair_body, 0)

        pltpu.sync_copy(acc_v, part_hbm.at[pl.ds(wid * n_neurons, n_neurons)])

    return sc_partials


def _lif_body(p_ref, v_ref, decay_ref, cf_ref, vth_ref, vreset_ref, out_ref):
    rec = jnp.sum(p_ref[...], axis=0)
    v = v_ref[...]
    decay = decay_ref[...]
    cf = cf_ref[...]
    vth = vth_ref[...]
    vreset = vreset_ref[...]
    new_v = decay * v + cf * rec
    v_scaled = (new_v - vth) / jnp.maximum(vth - vreset, 1e-6)
    new_z = (v_scaled > 0.0).astype(jnp.float32)
    out_ref[0] = new_z
    out_ref[1] = new_v - new_z * (vth - vreset)


def kernel(z, v, edge_index, weights, decay, current_factor, v_th, v_reset):
    batch, n = z.shape
    n_edges = weights.shape[0]
    chunk = 4000

    packed = jnp.bitwise_or(jnp.left_shift(edge_index[1], 16), edge_index[0])
    sc = _make_sc_partials(n, n_edges, batch, chunk)
    partials = sc(z.reshape(-1), packed, weights)
    partials = partials.reshape(_NW // batch, batch, n)

    d2 = decay.reshape(1, n)
    cf2 = current_factor.reshape(1, n)
    vth2 = v_th.reshape(1, n)
    vr2 = v_reset.reshape(1, n)
    return pl.pallas_call(
        _lif_body,
        out_shape=jax.ShapeDtypeStruct((2, batch, n), jnp.float32),
    )(partials, v, d2, cf2, vth2, vr2)


# speedup vs baseline: 1.6936x; 1.6936x over previous
import functools

import jax
import jax.numpy as jnp
from jax import lax
from jax.experimental import pallas as pl
from jax.experimental.pallas import tpu as pltpu
from jax.experimental.pallas import tpu_sc as plsc

_NC = 2
_NS = 16
_NW = _NC * _NS
_L = 16


def _make_sc_partials(n_neurons, n_pad, n_edges, batch, chunk):
    slices = _NW // batch
    epw = n_edges // slices
    chunks_pw = epw // chunk
    assert epw * slices == n_edges and chunks_pw * chunk == epw
    assert chunk % _L == 0 and chunk % 8 == 0 and chunks_pw % 2 == 0

    mesh = plsc.VectorSubcoreMesh(
        core_axis_name="c", subcore_axis_name="s",
        num_cores=_NC, num_subcores=_NS)

    @functools.partial(
        pl.kernel,
        out_type=jax.ShapeDtypeStruct((_NW * n_pad,), jnp.float32),
        mesh=mesh,
        scratch_types=[
            pltpu.VMEM((n_neurons,), jnp.float32),
            pltpu.VMEM((n_neurons,), jnp.float32),
            pltpu.VMEM((chunk,), jnp.int32),
            pltpu.VMEM((chunk,), jnp.int32),
            pltpu.VMEM((chunk,), jnp.float32),
            pltpu.VMEM((chunk,), jnp.float32),
            pltpu.SemaphoreType.DMA,
            pltpu.SemaphoreType.DMA,
            pltpu.SemaphoreType.DMA,
        ],
        compiler_params=pltpu.CompilerParams(needs_layout_passes=False),
    )
    def sc_partials(z_hbm, pp_hbm, w_hbm, part_hbm,
                    z_v, acc_v, pp0, pp1, w0, w1,
                    sem0, sem1, semz):
        wid = lax.axis_index("s") * _NC + lax.axis_index("c")
        b = wid % batch
        s = wid // batch
        zcopy = pltpu.async_copy(
            z_hbm.at[pl.ds(b * n_neurons, n_neurons)], z_v, semz)

        zero = jnp.zeros((_L,), jnp.float32)

        @plsc.parallel_loop(0, n_neurons // _L, unroll=8)
        def _(i):
            acc_v[pl.ds(i * _L, _L)] = zero
        zcopy.wait()

        base = s * chunks_pw
        bufs = ((pp0, w0), (pp1, w1))
        sems = (sem0, sem1)

        def start(g, slot):
            off = (base + g) * chunk
            pv, wv = bufs[slot]
            pltpu.async_copy(pp_hbm.at[pl.ds(off, chunk)], pv, sems[slot])
            pltpu.async_copy(w_hbm.at[pl.ds(off, chunk)], wv, sems[slot])

        def drain(g, slot):
            off = (base + g) * chunk
            pv, wv = bufs[slot]
            pltpu.make_async_copy(pp_hbm.at[pl.ds(off, chunk)], pv,
                                  sems[slot]).wait()
            pltpu.make_async_copy(w_hbm.at[pl.ds(off, chunk)], wv,
                                  sems[slot]).wait()

        start(0, 0)
        start(1, 1)

        def pair_body(gp, carry):
            for slot in range(2):
                g = gp * 2 + slot
                drain(g, slot)
                pv, wv = bufs[slot]

                @plsc.parallel_loop(0, chunk // _L, unroll=10)
                def _(j):
                    sl = pl.ds(j * _L, _L)
                    pp = pv[sl]
                    w = wv[sl]
                    pre = lax.shift_right_logical(pp, 16)
                    post = lax.bitwise_and(pp, 0xFFFF)
                    zg = plsc.load_gather(z_v, [pre])
                    plsc.addupdate_scatter(acc_v, [post], zg * w)

                @pl.when(g + 2 < chunks_pw)
                def _():
                    start(g + 2, slot)
            return carry
        lax.fori_loop(0, chunks_pw // 2, pair_body, 0)

        pltpu.sync_copy(acc_v, part_hbm.at[pl.ds(wid * n_pad, n_neurons)])

    return sc_partials


def _pack_body(e_ref, out_ref):
    out_ref[...] = jnp.bitwise_or(
        jnp.left_shift(e_ref[1], 16), e_ref[0])


def _lif_body(p_ref, v_ref, decay_ref, cf_ref, vth_ref, vreset_ref, out_ref):
    n = v_ref.shape[1]
    rec = jnp.sum(p_ref[...], axis=0)[:, :n]
    v = v_ref[...]
    decay = decay_ref[...]
    cf = cf_ref[...]
    vth = vth_ref[...]
    vreset = vreset_ref[...]
    new_v = decay * v + cf * rec
    v_scaled = (new_v - vth) / jnp.maximum(vth - vreset, 1e-6)
    new_z = (v_scaled > 0.0).astype(jnp.float32)
    out_ref[0] = new_z
    out_ref[1] = new_v - new_z * (vth - vreset)


def kernel(z, v, edge_index, weights, decay, current_factor, v_th, v_reset):
    batch, n = z.shape
    n_edges = weights.shape[0]
    chunk = 4000
    n_pad = -(-n // 128) * 128

    packed = pl.pallas_call(
        _pack_body,
        out_shape=jax.ShapeDtypeStruct((n_edges,), jnp.int32),
    )(edge_index)
    sc = _make_sc_partials(n, n_pad, n_edges, batch, chunk)
    partials = sc(z.reshape(-1), packed, weights)
    partials = partials.reshape(_NW // batch, batch, n_pad)

    d2 = decay.reshape(1, n)
    cf2 = current_factor.reshape(1, n)
    vth2 = v_th.reshape(1, n)
    vr2 = v_reset.reshape(1, n)
    return pl.pallas_call(
        _lif_body,
        out_shape=jax.ShapeDtypeStruct((2, batch, n), jnp.float32),
    )(partials, v, d2, cf2, vth2, vr2)


# speedup vs baseline: 1.7775x; 1.0495x over previous
import functools

import jax
import jax.numpy as jnp
from jax import lax
from jax.experimental import pallas as pl
from jax.experimental.pallas import tpu as pltpu
from jax.experimental.pallas import tpu_sc as plsc

_NC = 2
_NS = 16
_NW = _NC * _NS
_L = 16


def _make_sc_partials(n_neurons, n_pad, n_edges, batch, chunk):
    slices = _NW // batch
    epw = n_edges // slices
    chunks_pw = epw // chunk
    assert epw * slices == n_edges and chunks_pw * chunk == epw
    assert chunk % _L == 0 and chunk % 8 == 0 and chunks_pw % 2 == 0

    mesh = plsc.VectorSubcoreMesh(
        core_axis_name="c", subcore_axis_name="s",
        num_cores=_NC, num_subcores=_NS)

    @functools.partial(
        pl.kernel,
        out_type=jax.ShapeDtypeStruct((_NW * n_pad,), jnp.float32),
        mesh=mesh,
        scratch_types=[
            pltpu.VMEM((n_neurons,), jnp.float32),
            pltpu.VMEM((n_neurons,), jnp.float32),
            pltpu.VMEM((chunk,), jnp.int32),
            pltpu.VMEM((chunk,), jnp.int32),
            pltpu.VMEM((chunk,), jnp.float32),
            pltpu.VMEM((chunk,), jnp.float32),
            pltpu.SemaphoreType.DMA,
            pltpu.SemaphoreType.DMA,
            pltpu.SemaphoreType.DMA,
        ],
        compiler_params=pltpu.CompilerParams(needs_layout_passes=False),
    )
    def sc_partials(z_hbm, pp_hbm, w_hbm, part_hbm,
                    z_v, acc_v, pp0, pp1, w0, w1,
                    sem0, sem1, semz):
        wid = lax.axis_index("s") * _NC + lax.axis_index("c")
        b = wid % batch
        s = wid // batch
        zcopy = pltpu.async_copy(
            z_hbm.at[pl.ds(b * n_neurons, n_neurons)], z_v, semz)

        zero = jnp.zeros((_L,), jnp.float32)

        @plsc.parallel_loop(0, n_neurons // _L, unroll=8)
        def _(i):
            acc_v[pl.ds(i * _L, _L)] = zero
        zcopy.wait()

        base = s * chunks_pw
        bufs = ((pp0, w0), (pp1, w1))
        sems = (sem0, sem1)

        def start(g, slot):
            off = (base + g) * chunk
            pv, wv = bufs[slot]
            pltpu.async_copy(pp_hbm.at[pl.ds(off, chunk)], pv, sems[slot])
            pltpu.async_copy(w_hbm.at[pl.ds(off, chunk)], wv, sems[slot])

        def drain(g, slot):
            off = (base + g) * chunk
            pv, wv = bufs[slot]
            pltpu.make_async_copy(pp_hbm.at[pl.ds(off, chunk)], pv,
                                  sems[slot]).wait()
            pltpu.make_async_copy(w_hbm.at[pl.ds(off, chunk)], wv,
                                  sems[slot]).wait()

        start(0, 0)
        start(1, 1)

        def pair_body(gp, carry):
            for slot in range(2):
                g = gp * 2 + slot
                drain(g, slot)
                pv, wv = bufs[slot]

                @plsc.parallel_loop(0, chunk // _L, unroll=10)
                def _(j):
                    sl = pl.ds(j * _L, _L)
                    pp = pv[sl]
                    w = wv[sl]
                    pre = lax.shift_right_logical(pp, 16)
                    post = lax.bitwise_and(pp, 0xFFFF)
                    zg = plsc.load_gather(z_v, [pre])
                    plsc.addupdate_scatter(acc_v, [post], zg * w,
                                           mask=zg != 0.0)

                @pl.when(g + 2 < chunks_pw)
                def _():
                    start(g + 2, slot)
            return carry
        lax.fori_loop(0, chunks_pw // 2, pair_body, 0)

        row = b * slices + s
        pltpu.sync_copy(acc_v, part_hbm.at[pl.ds(row * n_pad, n_neurons)])

    return sc_partials


def _pack_body(e_ref, out_ref):
    out_ref[...] = jnp.bitwise_or(
        jnp.left_shift(e_ref[1], 16), e_ref[0])


def _lif_body(p_ref, v_ref, decay_ref, cf_ref, vth_ref, vreset_ref, out_ref):
    n = v_ref.shape[1]
    rec = jnp.sum(p_ref[...], axis=1)[:, :n]
    v = v_ref[...]
    decay = decay_ref[...]
    cf = cf_ref[...]
    vth = vth_ref[...]
    vreset = vreset_ref[...]
    new_v = decay * v + cf * rec
    v_scaled = (new_v - vth) / jnp.maximum(vth - vreset, 1e-6)
    new_z = (v_scaled > 0.0).astype(jnp.float32)
    out_ref[0] = new_z
    out_ref[1] = new_v - new_z * (vth - vreset)


def kernel(z, v, edge_index, weights, decay, current_factor, v_th, v_reset):
    batch, n = z.shape
    n_edges = weights.shape[0]
    chunk = 4000
    n_pad = -(-n // 128) * 128

    packed = pl.pallas_call(
        _pack_body,
        out_shape=jax.ShapeDtypeStruct((n_edges,), jnp.int32),
    )(edge_index)
    sc = _make_sc_partials(n, n_pad, n_edges, batch, chunk)
    partials = sc(z.reshape(-1), packed, weights)
    partials = partials.reshape(batch, _NW // batch, n_pad)

    d2 = decay.reshape(1, n)
    cf2 = current_factor.reshape(1, n)
    vth2 = v_th.reshape(1, n)
    vr2 = v_reset.reshape(1, n)
    return pl.pallas_call(
        _lif_body,
        out_shape=jax.ShapeDtypeStruct((2, batch, n), jnp.float32),
    )(partials, v, d2, cf2, vth2, vr2)
